# baseline (speedup 1.0000x reference)
import functools

import jax
import jax.numpy as jnp
from jax import lax
from jax.experimental import pallas as pl
from jax.experimental.pallas import tpu as pltpu
from jax.experimental.pallas import tpu_sc as plsc

B = 16384
D = 64
NC = 2
NS = 16
NW = NC * NS
BPW = B // NW
CH = 128
NCHUNK = BPW // CH


@functools.partial(
    pl.kernel,
    mesh=plsc.VectorSubcoreMesh(core_axis_name="c", subcore_axis_name="s"),
    compiler_params=pltpu.CompilerParams(needs_layout_passes=False,
                                         use_tc_tiling_on_sc=False),
    out_type=jax.ShapeDtypeStruct((B,), jnp.float32),
    scratch_types=[
        pltpu.VMEM((BPW,), jnp.int32),
        pltpu.VMEM((BPW,), jnp.int32),
        pltpu.VMEM((BPW,), jnp.int32),
        pltpu.VMEM((CH, D), jnp.float32),
        pltpu.VMEM((CH, D), jnp.float32),
        pltpu.VMEM((CH, D), jnp.float32),
        pltpu.VMEM((CH, D), jnp.float32),
        pltpu.VMEM((CH, D), jnp.float32),
        pltpu.VMEM((CH, D), jnp.float32),
        pltpu.VMEM((256,), jnp.float32),
        pltpu.VMEM((BPW,), jnp.float32),
        pltpu.SemaphoreType.DMA,
    ],
)
def _sc_score(heads, rels, tails, ere, eim, rre, rim, out,
              hidx, ridx, tidx, hre, him, rrev, rimv, tre, tim, tr, outv,
              sem):
    wid = lax.axis_index("s") * NC + lax.axis_index("c")
    base = wid * BPW
    pltpu.sync_copy(heads.at[pl.ds(base, BPW)], hidx)
    pltpu.sync_copy(rels.at[pl.ds(base, BPW)], ridx)
    pltpu.sync_copy(tails.at[pl.ds(base, BPW)], tidx)
    for c in range(NCHUNK):
        csl = pl.ds(c * CH, CH)
        cps = [
            pltpu.async_copy(ere.at[hidx.at[csl]], hre, sem),
            pltpu.async_copy(eim.at[hidx.at[csl]], him, sem),
            pltpu.async_copy(rre.at[ridx.at[csl]], rrev, sem),
            pltpu.async_copy(rim.at[ridx.at[csl]], rimv, sem),
            pltpu.async_copy(ere.at[tidx.at[csl]], tre, sem),
            pltpu.async_copy(eim.at[tidx.at[csl]], tim, sem),
        ]
        for cp in cps:
            cp.wait()

        def group(g, carry):
            rows = g * 16 + lax.iota(jnp.int32, 16)
            acc = jnp.zeros((16,), jnp.float32)
            for d in range(D):
                col = jnp.full((16,), d, jnp.int32)
                h_re = plsc.load_gather(hre, [rows, col])
                h_im = plsc.load_gather(him, [rows, col])
                r_re = plsc.load_gather(rrev, [rows, col])
                r_im = plsc.load_gather(rimv, [rows, col])
                t_re = plsc.load_gather(tre, [rows, col])
                t_im = plsc.load_gather(tim, [rows, col])
                acc = (acc
                       + (h_re * r_re - h_im * r_im) * t_re
                       + (h_re * r_im + h_im * r_re) * t_im)
            outv[pl.ds(c * CH + g * 16, 16)] = acc
            return carry

        lax.fori_loop(0, CH // 16, group, 0)
    pltpu.sync_copy(outv, out.at[pl.ds(base, BPW)])


def kernel(heads, relations, tails, entity_re, entity_im, relation_re,
           relation_im):
    return _sc_score(heads.astype(jnp.int32), relations.astype(jnp.int32),
                     tails.astype(jnp.int32), entity_re, entity_im,
                     relation_re, relation_im)


# speedup vs baseline: 2.0717x; 2.0717x over previous
import functools

import jax
import jax.numpy as jnp
from jax import lax
from jax.experimental import pallas as pl
from jax.experimental.pallas import tpu as pltpu
from jax.experimental.pallas import tpu_sc as plsc

B = 16384
D = 64
NC = 2
NS = 16
NW = NC * NS
BPW = B // NW
CHT = 16
NCHUNK = BPW // CHT
ETILES = 1000000 // 8


@functools.partial(
    pl.kernel,
    mesh=plsc.VectorSubcoreMesh(core_axis_name="c", subcore_axis_name="s"),
    compiler_params=pltpu.CompilerParams(needs_layout_passes=False,
                                         use_tc_tiling_on_sc=True),
    out_type=jax.ShapeDtypeStruct((B,), jnp.float32),
    scratch_types=[
        pltpu.VMEM((BPW,), jnp.int32),
        pltpu.VMEM((BPW,), jnp.int32),
        pltpu.VMEM((BPW,), jnp.int32),
        pltpu.VMEM((CHT, D), jnp.float32),
        pltpu.VMEM((CHT, D), jnp.float32),
        pltpu.VMEM((CHT, D), jnp.float32),
        pltpu.VMEM((CHT, D), jnp.float32),
        pltpu.VMEM((CHT, 2 * D), jnp.float32),
        pltpu.VMEM((BPW,), jnp.float32),
        pltpu.SemaphoreType.DMA,
    ],
)
def _sc_score(heads, rels, tails, ere3, eim3, relcat, out,
              hidx, ridx, tidx, hre_r, him_r, tre_r, tim_r, rel_t, outv,
              sem):
    wid = lax.axis_index("s") * NC + lax.axis_index("c")
    base = wid * BPW
    pltpu.sync_copy(heads.at[pl.ds(base, BPW)], hidx)
    pltpu.sync_copy(rels.at[pl.ds(base, BPW)], ridx)
    pltpu.sync_copy(tails.at[pl.ds(base, BPW)], tidx)
    lanes = lax.iota(jnp.int32, CHT)

    def chunk(c, carry):
        hv = hidx[pl.ds(c * CHT, CHT)]
        tv = tidx[pl.ds(c * CHT, CHT)]
        ht = lax.shift_right_logical(hv, 3)
        tt = lax.shift_right_logical(tv, 3)
        hs = lax.bitwise_and(hv, 7)
        ts = lax.bitwise_and(tv, 7)
        cprel = pltpu.async_copy(relcat.at[ridx.at[pl.ds(c * CHT, CHT)]],
                                 rel_t, sem)
        cps = []
        for i in range(CHT):
            cps.append(pltpu.async_copy(ere3.at[ht[i], hs[i]],
                                        hre_r.at[i], sem))
            cps.append(pltpu.async_copy(eim3.at[ht[i], hs[i]],
                                        him_r.at[i], sem))
            cps.append(pltpu.async_copy(ere3.at[tt[i], ts[i]],
                                        tre_r.at[i], sem))
            cps.append(pltpu.async_copy(eim3.at[tt[i], ts[i]],
                                        tim_r.at[i], sem))
        for cp in cps:
            cp.wait()
        cprel.wait()
        acc = jnp.zeros((CHT,), jnp.float32)
        col = jnp.zeros((CHT,), jnp.int32)
        one = jnp.ones((CHT,), jnp.int32)
        for d in range(D):
            h_re = plsc.load_gather(hre_r, [lanes, col])
            h_im = plsc.load_gather(him_r, [lanes, col])
            t_re = plsc.load_gather(tre_r, [lanes, col])
            t_im = plsc.load_gather(tim_r, [lanes, col])
            r_re = plsc.load_gather(rel_t, [lanes, col])
            r_im = plsc.load_gather(rel_t, [lanes, col + D])
            acc = (acc
                   + (h_re * r_re - h_im * r_im) * t_re
                   + (h_re * r_im + h_im * r_re) * t_im)
            col = col + one
        outv[pl.ds(c * CHT, CHT)] = acc
        return carry

    lax.fori_loop(0, NCHUNK, chunk, 0)
    pltpu.sync_copy(outv, out.at[pl.ds(base, BPW)])


def kernel(heads, relations, tails, entity_re, entity_im, relation_re,
           relation_im):
    ere3 = entity_re.reshape(ETILES, 8, D)
    eim3 = entity_im.reshape(ETILES, 8, D)
    relcat = jnp.concatenate([relation_re, relation_im], axis=1)
    return _sc_score(heads.astype(jnp.int32), relations.astype(jnp.int32),
                     tails.astype(jnp.int32), ere3, eim3, relcat)


# speedup vs baseline: 2.1213x; 1.0239x over previous
import functools

import jax
import jax.numpy as jnp
from jax import lax
from jax.experimental import pallas as pl
from jax.experimental.pallas import tpu as pltpu
from jax.experimental.pallas import tpu_sc as plsc

B = 16384
D = 64
NC = 2
NS = 16
NW = NC * NS
BPW = B // NW
CHT = 16
NCHUNK = BPW // CHT
ETILES = 1000000 // 8

_ROWBUF = pltpu.VMEM((2, 8, D), jnp.float32)


@functools.partial(
    pl.kernel,
    mesh=plsc.VectorSubcoreMesh(core_axis_name="c", subcore_axis_name="s"),
    compiler_params=pltpu.CompilerParams(needs_layout_passes=False,
                                         use_tc_tiling_on_sc=True),
    out_type=jax.ShapeDtypeStruct((B,), jnp.float32),
    scratch_types=[
        pltpu.VMEM((BPW,), jnp.int32),
        pltpu.VMEM((BPW,), jnp.int32),
        pltpu.VMEM((BPW,), jnp.int32),
        _ROWBUF, _ROWBUF, _ROWBUF, _ROWBUF,
        pltpu.VMEM((CHT, 2 * D), jnp.float32),
        _ROWBUF, _ROWBUF, _ROWBUF, _ROWBUF,
        pltpu.VMEM((CHT, 2 * D), jnp.float32),
        pltpu.VMEM((BPW,), jnp.float32),
        pltpu.SemaphoreType.DMA,
        pltpu.SemaphoreType.DMA,
    ],
)
def _sc_score(heads, rels, tails, ere3, eim3, relcat, out,
              hidx, ridx, tidx,
              hreA, himA, treA, timA, relA,
              hreB, himB, treB, timB, relB,
              outv, semA, semB):
    wid = lax.axis_index("s") * NC + lax.axis_index("c")
    base = wid * BPW
    pltpu.sync_copy(heads.at[pl.ds(base, BPW)], hidx)
    pltpu.sync_copy(rels.at[pl.ds(base, BPW)], ridx)
    pltpu.sync_copy(tails.at[pl.ds(base, BPW)], tidx)
    lanes = lax.iota(jnp.int32, CHT)
    lhi = lax.shift_right_logical(lanes, 3)
    llo = lax.bitwise_and(lanes, 7)

    def fire(c, bufs, sem):
        hre_x, him_x, tre_x, tim_x, rel_x = bufs
        hv = hidx[pl.ds(c * CHT, CHT)]
        tv = tidx[pl.ds(c * CHT, CHT)]
        ht = lax.shift_right_logical(hv, 3)
        tt = lax.shift_right_logical(tv, 3)
        hs = lax.bitwise_and(hv, 7)
        ts = lax.bitwise_and(tv, 7)
        pltpu.async_copy(relcat.at[ridx.at[pl.ds(c * CHT, CHT)]],
                         rel_x, sem)
        for i in range(CHT):
            dst = (i // 8, i % 8)
            pltpu.async_copy(ere3.at[ht[i], hs[i]], hre_x.at[dst], sem)
            pltpu.async_copy(eim3.at[ht[i], hs[i]], him_x.at[dst], sem)
            pltpu.async_copy(ere3.at[tt[i], ts[i]], tre_x.at[dst], sem)
            pltpu.async_copy(eim3.at[tt[i], ts[i]], tim_x.at[dst], sem)

    def drain(bufs, sem):
        hre_x, him_x, tre_x, tim_x, rel_x = bufs
        for buf in (hre_x, him_x, tre_x, tim_x):
            for i in range(CHT):
                pltpu.make_async_copy(ere3.at[0, 0],
                                      buf.at[i // 8, i % 8], sem).wait()
        pltpu.make_async_copy(relcat.at[pl.ds(0, CHT)], rel_x, sem).wait()

    def compute(c, bufs):
        hre_x, him_x, tre_x, tim_x, rel_x = bufs
        acc = jnp.zeros((CHT,), jnp.float32)
        col = jnp.zeros((CHT,), jnp.int32)
        one = jnp.ones((CHT,), jnp.int32)
        for d in range(D):
            h_re = plsc.load_gather(hre_x, [lhi, llo, col])
            h_im = plsc.load_gather(him_x, [lhi, llo, col])
            t_re = plsc.load_gather(tre_x, [lhi, llo, col])
            t_im = plsc.load_gather(tim_x, [lhi, llo, col])
            r_re = plsc.load_gather(rel_x, [lanes, col])
            r_im = plsc.load_gather(rel_x, [lanes, col + D])
            acc = (acc
                   + (h_re * r_re - h_im * r_im) * t_re
                   + (h_re * r_im + h_im * r_re) * t_im)
            col = col + one
        outv[pl.ds(c * CHT, CHT)] = acc

    bufsA = (hreA, himA, treA, timA, relA)
    bufsB = (hreB, himB, treB, timB, relB)
    fire(0, bufsA, semA)
    fire(1, bufsB, semB)

    def body(i, carry):
        c0 = 2 * i
        drain(bufsA, semA)
        compute(c0, bufsA)
        fire(c0 + 2, bufsA, semA)
        drain(bufsB, semB)
        compute(c0 + 1, bufsB)
        fire(c0 + 3, bufsB, semB)
        return carry

    lax.fori_loop(0, NCHUNK // 2 - 1, body, 0)
    drain(bufsA, semA)
    compute(NCHUNK - 2, bufsA)
    drain(bufsB, semB)
    compute(NCHUNK - 1, bufsB)
    pltpu.sync_copy(outv, out.at[pl.ds(base, BPW)])


def kernel(heads, relations, tails, entity_re, entity_im, relation_re,
           relation_im):
    ere3 = entity_re.reshape(ETILES, 8, D)
    eim3 = entity_im.reshape(ETILES, 8, D)
    relcat = jnp.concatenate([relation_re, relation_im], axis=1)
    return _sc_score(heads.astype(jnp.int32), relations.astype(jnp.int32),
                     tails.astype(jnp.int32), ere3, eim3, relcat)
